# baseline (speedup 1.0000x reference)
import functools

import jax
import jax.numpy as jnp
from jax import lax
from jax.experimental import pallas as pl
from jax.experimental.pallas import tpu as pltpu
from jax.experimental.pallas import tpu_sc as plsc

_BLK = 128


def _build_gather(B, D, B0, NC, NW, per_w):
    n_blk = per_w // _BLK
    n_pair = n_blk // 2
    mesh = plsc.VectorSubcoreMesh(core_axis_name="c", subcore_axis_name="s")

    @functools.partial(
        pl.kernel,
        mesh=mesh,
        out_type=jax.ShapeDtypeStruct((B // B0, D, B0), jnp.float32),
        scratch_types=[
            pltpu.VMEM((per_w,), jnp.int32),
            pltpu.VMEM((2, _BLK), jnp.int32),
            pltpu.VMEM((2, _BLK, 2 * D), jnp.float32),
            pltpu.VMEM((2, D, _BLK), jnp.float32),
            pltpu.SemaphoreType.DMA,
            pltpu.SemaphoreType.DMA,
            pltpu.SemaphoreType.DMA,
            pltpu.SemaphoreType.DMA,
        ],
        compiler_params=pltpu.CompilerParams(needs_layout_passes=False),
    )
    def gather_kernel(ids_hbm, tab_hbm, out_hbm, idx_v, gidx, buf, obuf,
                      sg0, sg1, so0, so1):
        sg = (sg0, sg1)
        so = (so0, so1)
        wid = lax.axis_index("s") * NC + lax.axis_index("c")
        p0 = pl.multiple_of(wid * per_w, 8)
        iota = lax.iota(jnp.int32, 16)

        pltpu.sync_copy(ids_hbm.at[pl.ds(p0, per_w)], idx_v)

        def start_gather(k, q):
            g_q = gidx.at[q]
            for t in range(8):
                g_q[pl.ds(t * 16, 16)] = idx_v[pl.ds(k * _BLK + t * 16, 16)]
            pltpu.async_copy(tab_hbm.at[g_q], buf.at[q], sg[q])

        def wait_gather(q):
            pltpu.make_async_copy(
                tab_hbm.at[gidx.at[q]], buf.at[q], sg[q]
            ).wait()

        def extract(k, q):
            buf_q = buf.at[q]
            obuf_q = obuf.at[q]
            rows = [iota + t * 16 for t in range(8)]

            def jbody(j, c):
                jv = jnp.full((16,), 0, jnp.int32) + j
                for t in range(8):
                    vals = plsc.load_gather(buf_q, [rows[t], jv])
                    obuf_q[j, pl.ds(t * 16, 16)] = vals
                return c

            lax.fori_loop(0, D, jbody, 0)

        def start_write(k, q):
            pp = p0 + k * _BLK
            s = pp // B0
            b0 = pl.multiple_of(pp % B0, _BLK)
            pltpu.async_copy(
                obuf.at[q], out_hbm.at[s, :, pl.ds(b0, _BLK)], so[q]
            )

        def wait_write(q):
            pltpu.make_async_copy(
                obuf.at[q], out_hbm.at[0, :, pl.ds(0, _BLK)], so[q]
            ).wait()

        start_gather(0, 0)
        start_gather(1, 1)

        def body(jp, carry):
            for q in range(2):
                k = jp * 2 + q
                wait_gather(q)

                @pl.when(jp > 0)
                def _():
                    wait_write(q)

                extract(k, q)
                start_write(k, q)

                @pl.when(jp < n_pair - 1)
                def _():
                    start_gather(k + 2, q)

            return carry

        lax.fori_loop(0, n_pair, body, 0)
        wait_write(0)
        wait_write(1)

    return gather_kernel


def kernel(token_ids, embedding):
    B0, S = token_ids.shape
    V, D = embedding.shape
    B = B0 * S
    flat_ids = token_ids.T.reshape(B).astype(jnp.int32)
    tab2 = jnp.pad(embedding, ((0, 0), (0, D)))

    info = plsc.get_sparse_core_info()
    NC, NS = info.num_cores, info.num_subcores
    NW = NC * NS
    per_w = B // NW

    out3 = _build_gather(B, D, B0, NC, NW, per_w)(flat_ids, tab2)
    return out3.transpose(2, 0, 1)


# speedup vs baseline: 1.4685x; 1.4685x over previous
import functools

import jax
import jax.numpy as jnp
from jax import lax
from jax.experimental import pallas as pl
from jax.experimental.pallas import tpu as pltpu
from jax.experimental.pallas import tpu_sc as plsc


def _build_gather(B, D, C, NC, NW, b_per_w):
    n_chunks = b_per_w // C
    n_pairs = n_chunks // 2
    mesh = plsc.VectorSubcoreMesh(core_axis_name="c", subcore_axis_name="s")

    @functools.partial(
        pl.kernel,
        mesh=mesh,
        out_type=jax.ShapeDtypeStruct((B, D), jnp.float32),
        scratch_types=[
            pltpu.VMEM((b_per_w,), jnp.int32),
            pltpu.VMEM((2, C, D), jnp.float32),
            pltpu.SemaphoreType.DMA,
            pltpu.SemaphoreType.DMA,
        ],
        compiler_params=pltpu.CompilerParams(use_tc_tiling_on_sc=False),
    )
    def gather_kernel(ids_hbm, table_hbm, out_hbm, idx_v, rows_v, sem0, sem1):
        sems = (sem0, sem1)
        wid = lax.axis_index("s") * NC + lax.axis_index("c")
        base = pl.multiple_of(wid * b_per_w, 8)

        pltpu.sync_copy(ids_hbm.at[pl.ds(base, b_per_w)], idx_v)

        def start(i, b):
            pltpu.async_copy(
                table_hbm.at[idx_v.at[pl.ds(i * C, C)]], rows_v.at[b], sems[b]
            )

        def finish(i, b):
            pltpu.make_async_copy(
                table_hbm.at[idx_v.at[pl.ds(i * C, C)]], rows_v.at[b], sems[b]
            ).wait()
            off = pl.multiple_of(base + i * C, 8)
            pltpu.sync_copy(rows_v.at[b], out_hbm.at[pl.ds(off, C)])

        start(0, 0)
        start(1, 1)

        def body(j, carry):
            for b in range(2):
                i = j * 2 + b
                finish(i - 2, b)
                start(i, b)
            return carry

        lax.fori_loop(1, n_pairs, body, 0)
        finish(n_chunks - 2, 0)
        finish(n_chunks - 1, 1)

    return gather_kernel


def kernel(token_ids, embedding):
    B0, S = token_ids.shape
    D = embedding.shape[1]
    B = B0 * S
    flat_ids = token_ids.T.reshape(B).astype(jnp.int32)

    info = plsc.get_sparse_core_info()
    NC, NS = info.num_cores, info.num_subcores
    NW = NC * NS
    b_per_w = B // NW
    C = 800

    out = _build_gather(B, D, C, NC, NW, b_per_w)(flat_ids, embedding)
    return out.reshape(S, B0, D).transpose(1, 0, 2)
